# baseline (speedup 1.0000x reference)
import math

import jax
import jax.numpy as jnp
from jax import lax
from jax.experimental import pallas as pl
from jax.experimental.pallas import tpu as pltpu
from jax.experimental.pallas import tpu_sc as plsc

B, L, DK, R, BL, NB = 32, 2048, 64, 4, 64, 32
NEG_BIG = -1e9
NEG_SELF = -1e5


def _eye64():
    i = lax.broadcasted_iota(jnp.int32, (64, 64), 0)
    j = lax.broadcasted_iota(jnp.int32, (64, 64), 1)
    return (i == j).astype(jnp.float32)


def _k1_body(q_ref, rm_ref, rank_ref, meta_ref):
    nq = q_ref[0]
    rm = rm_ref[0]
    rmn = rm / jnp.sqrt(jnp.sum(rm * rm, axis=0, keepdims=True))
    mm = jnp.dot(nq.astype(jnp.bfloat16), rmn.astype(jnp.bfloat16),
                 preferred_element_type=jnp.float32)

    iota_l = lax.broadcasted_iota(jnp.int32, (L, NB), 1)
    ri = lax.broadcasted_iota(jnp.int32, (128, 128), 0)
    ci = lax.broadcasted_iota(jnp.int32, (128, 128), 1)
    tril128 = (ci <= ri).astype(jnp.float32)
    cols = []
    for s in range(R):
        ms = mm[:, s * 16:(s + 1) * 16]
        sc = jnp.concatenate([ms, -ms], axis=1)
        mx = jnp.max(sc, axis=1, keepdims=True)
        h = jnp.min(jnp.where(sc == mx, iota_l, NB), axis=1, keepdims=True)
        onehot = (iota_l == h).astype(jnp.float32)
        carry = jnp.zeros((1, NB), jnp.float32)
        parts = []
        for k in range(L // 128):
            blk = onehot[k * 128:(k + 1) * 128]
            cs = jnp.dot(tril128, blk, preferred_element_type=jnp.float32) + carry
            parts.append(cs)
            carry = cs[127:128, :]
        csum = jnp.concatenate(parts, axis=0)
        incl = carry
        for sh in (1, 2, 4, 8, 16):
            incl = incl + jnp.concatenate(
                [jnp.zeros((1, sh), jnp.float32), incl[:, :NB - sh]], axis=1)
        start = incl - carry
        rank_f = (jnp.sum(csum * onehot, axis=1, keepdims=True) - 1.0
                  + jnp.sum(onehot * start, axis=1, keepdims=True))
        rank_ref[0, s] = rank_f.astype(jnp.int32)
        cols.append(h.astype(jnp.float32))
        cols.append(jnp.floor(rank_f / BL))

    cols8 = jnp.concatenate(cols, axis=1)
    eye = _eye64()
    partsT = []
    for k in range(L // 64):
        blk = cols8[k * 64:(k + 1) * 64]
        partsT.append(lax.dot_general(blk, eye, (((0,), (0,)), ((), ())),
                                      preferred_element_type=jnp.float32))
    rows8 = jnp.concatenate(partsT, axis=1)
    meta_ref[0, 0:1, :] = lax.broadcasted_iota(jnp.int32, (1, L), 1)
    meta_ref[0, 1:9, :] = rows8.astype(jnp.int32)


def _run_k1(query, rm2d):
    return pl.pallas_call(
        _k1_body,
        grid=(B,),
        in_specs=[
            pl.BlockSpec((1, L, DK), lambda b: (b, 0, 0)),
            pl.BlockSpec((1, DK, R * 16), lambda b: (b, 0, 0)),
        ],
        out_specs=[
            pl.BlockSpec((1, R, L, 1), lambda b: (b, 0, 0, 0)),
            pl.BlockSpec((1, 9, L), lambda b: (b, 0, 0)),
        ],
        out_shape=[
            jax.ShapeDtypeStruct((B, R, L, 1), jnp.int32),
            jax.ShapeDtypeStruct((B, 9, L), jnp.int32),
        ],
    )(query, rm2d)


def _k2_body(qv_hbm, meta_hbm, rank_hbm, qvs_hbm, ms_hbm, qc_hbm,
             rkfull, idxg, qvbuf, min0, min1, min2, min3, min4, min5,
             mo0, mo1, mo2, mo3, mo4, mo5, qcbuf, sem):
    wid = lax.axis_index("s") * 2 + lax.axis_index("c")
    mins = [min0, min1, min2, min3, min4, min5]
    mouts = [mo0, mo1, mo2, mo3, mo4, mo5]

    def pair_step(t, _):
        pair = wid * 4 + t
        b = pair // R
        s_ = pair % R
        base = pair * L
        pltpu.sync_copy(rank_hbm.at[pl.ds(base, L)], rkfull)
        pltpu.sync_copy(meta_hbm.at[pl.ds(b * 9 * L, L)], mins[0])
        pltpu.sync_copy(meta_hbm.at[pl.ds((b * 9 + 1 + 2 * s_) * L, L)], mins[1])
        for c in range(R):
            pltpu.sync_copy(meta_hbm.at[pl.ds((b * 9 + 2 + 2 * c) * L, L)],
                            mins[2 + c])

        def mstep(i, _):
            idx = rkfull[pl.ds(i * 16, 16)]
            idx8 = idx * 8
            for cc in range(6):
                vals = mins[cc][pl.ds(i * 16, 16)]
                plsc.store_scatter(mouts[cc], [idx], vals)
                plsc.store_scatter(qcbuf, [idx8 + cc], vals)
            return 0
        lax.fori_loop(0, L // 16, mstep, 0)
        for cc in range(6):
            pltpu.sync_copy(mouts[cc], ms_hbm.at[pl.ds((pair * 6 + cc) * L, L)])
        pltpu.sync_copy(qcbuf, qc_hbm.at[pl.ds(pair * L * 8, L * 8)])

        for c4 in range(4):
            l0 = c4 * 512
            pltpu.sync_copy(qv_hbm.at[b, pl.ds(l0, 512)], qvbuf)
            for i in range(32):
                row, off = i // 8, (i % 8) * 16
                idxg[row, pl.ds(off, 16)] = rkfull[pl.ds(l0 + i * 16, 16)] + base
            handles = []
            for j in range(4):
                handles.append(pltpu.async_copy(
                    qvbuf.at[pl.ds(j * 128, 128)], qvs_hbm.at[idxg.at[j]], sem))
            for hcp in handles:
                hcp.wait()
        return 0

    lax.fori_loop(0, 4, pair_step, 0)


def _run_k2(qv, meta, rank_flat):
    mesh = plsc.VectorSubcoreMesh(core_axis_name="c", subcore_axis_name="s")
    fn = pl.kernel(
        _k2_body,
        out_type=[
            jax.ShapeDtypeStruct((B * R * L, 2 * DK), jnp.float32),
            jax.ShapeDtypeStruct((B * R * 6 * L,), jnp.int32),
            jax.ShapeDtypeStruct((B * R * L * 8,), jnp.int32),
        ],
        mesh=mesh,
        compiler_params=pltpu.CompilerParams(needs_layout_passes=False),
        scratch_types=(
            [pltpu.VMEM((L,), jnp.int32),
             pltpu.VMEM((4, 128), jnp.int32),
             pltpu.VMEM((512, 2 * DK), jnp.float32)]
            + [pltpu.VMEM((L,), jnp.int32) for _ in range(12)]
            + [pltpu.VMEM((L * 8,), jnp.int32)]
            + [pltpu.SemaphoreType.DMA]
        ),
    )
    return fn(qv, meta, rank_flat)


def _k3_body(qvs_ref, ms_ref, qc_ref, att_ref, qk_scr):
    qvs = qvs_ref[0, 0]
    qs = qvs[:, 0:DK]
    vs = qvs[:, DK:2 * DK]
    mrows = ms_ref[0, 0].astype(jnp.float32)

    kn = qs / jnp.maximum(jnp.sqrt(jnp.sum(qs * qs, axis=1, keepdims=True)), 1e-12)
    kall = jnp.concatenate([kn[L - BL:], kn], axis=0)
    vall = jnp.concatenate([vs[L - BL:], vs], axis=0)
    mcat = jnp.concatenate([mrows[:, L - BL:], mrows], axis=1)
    qcolf = qc_ref[0, 0].astype(jnp.float32)

    inv_sqrt = 1.0 / math.sqrt(DK)
    for n in range(NB):
        qtile = qs[n * BL:(n + 1) * BL]
        ktile = kall[n * BL:n * BL + 2 * BL]
        qk_scr[n * BL:(n + 1) * BL, :] = lax.dot_general(
            qtile, ktile, (((1,), (1,)), ((), ())),
            preferred_element_type=jnp.float32) * inv_sqrt

    CH = 256
    for u in range(L // CH):
        r0 = u * CH
        qk = qk_scr[r0:r0 + CH, :]
        kband = []
        for c in range(6):
            kband.append(jnp.concatenate(
                [jnp.broadcast_to(mcat[c:c + 1, m * BL:m * BL + 2 * BL], (BL, 2 * BL))
                 for m in range(u * 4, u * 4 + 4)], axis=0))
        qcol = qcolf[r0:r0 + CH]
        qi, shq = qcol[:, 0:1], qcol[:, 1:2]
        qk = jnp.where(shq != kband[1], NEG_BIG, qk)
        qk = jnp.where(qi < kband[0], NEG_BIG, qk)
        qk = jnp.where(qi == kband[0], NEG_SELF, qk)
        ck = jnp.zeros((CH, 2 * BL), jnp.float32)
        for sp in range(R):
            bqc = qcol[:, 2 + sp:3 + sp]
            prv = jnp.where(bqc == 0.0, float(NB - 1), bqc - 1.0)
            ck = (ck + (kband[2 + sp] == bqc).astype(jnp.float32)
                  + (kband[2 + sp] == prv).astype(jnp.float32))
        m = jnp.max(qk, axis=1, keepdims=True)
        p = jnp.exp(qk - m)
        ssum = jnp.sum(p, axis=1, keepdims=True)
        qk_scr[r0:r0 + CH, :] = p / (ssum * ck)
        att_ref[0, 0, r0:r0 + CH, DK:DK + 1] = m + jnp.log(ssum)
        att_ref[0, 0, r0:r0 + CH, DK + 1:2 * DK] = jnp.zeros(
            (CH, DK - 1), jnp.float32)

    for n in range(NB):
        sm_t = qk_scr[n * BL:(n + 1) * BL, :]
        vtile = vall[n * BL:n * BL + 2 * BL]
        att_ref[0, 0, n * BL:(n + 1) * BL, 0:DK] = jnp.dot(
            sm_t, vtile, preferred_element_type=jnp.float32)


def _run_k3(qvs, ms, qc):
    return pl.pallas_call(
        _k3_body,
        grid=(B, R),
        in_specs=[
            pl.BlockSpec((1, 1, L, 2 * DK), lambda b, s: (b, s, 0, 0)),
            pl.BlockSpec((1, 1, 6, L), lambda b, s: (b, s, 0, 0)),
            pl.BlockSpec((1, 1, L, 8), lambda b, s: (b, s, 0, 0)),
        ],
        out_specs=pl.BlockSpec((1, 1, L, 2 * DK), lambda b, s: (b, s, 0, 0)),
        out_shape=jax.ShapeDtypeStruct((B, R, L, 2 * DK), jnp.float32),
        scratch_shapes=[pltpu.VMEM((L, 2 * BL), jnp.float32)],
    )(qvs, ms, qc)


def _k4_body(att_hbm, rank_hbm, atto_hbm, rkfull, idxg, abuf, sem):
    wid = lax.axis_index("s") * 2 + lax.axis_index("c")

    def step(it, _):
        t = it // 4
        c4 = it % 4
        pair = wid * 4 + t
        b = pair // R
        s_ = pair % R
        l0 = c4 * 512
        base = pair * L
        pltpu.sync_copy(rank_hbm.at[pl.ds(base + l0, 512)], rkfull)
        for i in range(32):
            row, off = i // 8, (i % 8) * 16
            idxg[row, pl.ds(off, 16)] = rkfull[pl.ds(i * 16, 16)] + base
        handles = []
        for j in range(4):
            handles.append(pltpu.async_copy(
                att_hbm.at[idxg.at[j]], abuf.at[pl.ds(j * 128, 128)], sem))
        for hcp in handles:
            hcp.wait()
        pltpu.sync_copy(abuf, atto_hbm.at[b, s_, pl.ds(l0, 512)])
        return 0

    lax.fori_loop(0, 16, step, 0)


def _run_k4(att_flat, rank_flat):
    mesh = plsc.VectorSubcoreMesh(core_axis_name="c", subcore_axis_name="s")
    fn = pl.kernel(
        _k4_body,
        out_type=jax.ShapeDtypeStruct((B, R, L, 2 * DK), jnp.float32),
        mesh=mesh,
        compiler_params=pltpu.CompilerParams(needs_layout_passes=False),
        scratch_types=[
            pltpu.VMEM((512,), jnp.int32),
            pltpu.VMEM((4, 128), jnp.int32),
            pltpu.VMEM((512, 2 * DK), jnp.float32),
            pltpu.SemaphoreType.DMA,
        ],
    )
    return fn(att_flat, rank_flat)


def _k5_body(att_ref, out_ref):
    acc = jnp.zeros((L, DK), jnp.float32)
    for s in range(R):
        col = att_ref[0, s, :, DK:DK + 1]
        m = jnp.max(col, axis=0, keepdims=True)
        e = jnp.exp(col - m)
        w = e / jnp.sum(e, axis=0, keepdims=True)
        acc = acc + w * att_ref[0, s, :, 0:DK]
    out_ref[0] = acc


def _run_k5(att_o):
    return pl.pallas_call(
        _k5_body,
        grid=(B,),
        in_specs=[pl.BlockSpec((1, R, L, 2 * DK), lambda b: (b, 0, 0, 0))],
        out_specs=pl.BlockSpec((1, L, DK), lambda b: (b, 0, 0)),
        out_shape=jax.ShapeDtypeStruct((B, L, DK), jnp.float32),
    )(att_o)


def kernel(query, value, seed, rand_matrix):
    rm2d = rand_matrix.reshape(B, DK, R * 16)
    qv = jnp.concatenate([query, value], axis=-1)
    nq = query / jnp.maximum(
        jnp.sqrt(jnp.sum(query * query, axis=-1, keepdims=True)), 1e-12)
    rank, meta = _run_k1(nq, rm2d)
    rank_flat = rank.reshape(B * R * L)
    qvs, ms, qc = _run_k2(qv, meta.reshape(B * 9 * L), rank_flat)
    att = _run_k3(qvs.reshape(B, R, L, 2 * DK), ms.reshape(B, R, 6, L),
                  qc.reshape(B, R, L, 8))
    att_o = _run_k4(att.reshape(B * R * L, 2 * DK), rank_flat)
    return _run_k5(att_o)


# speedup vs baseline: 1.0603x; 1.0603x over previous
import math

import jax
import jax.numpy as jnp
from jax import lax
from jax.experimental import pallas as pl
from jax.experimental.pallas import tpu as pltpu
from jax.experimental.pallas import tpu_sc as plsc

B, L, DK, R, BL, NB = 32, 2048, 64, 4, 64, 32
NEG_BIG = -1e9
NEG_SELF = -1e5


def _eye64():
    i = lax.broadcasted_iota(jnp.int32, (64, 64), 0)
    j = lax.broadcasted_iota(jnp.int32, (64, 64), 1)
    return (i == j).astype(jnp.float32)


def _k1_body(q_ref, rm_ref, rank_ref, meta_ref):
    nq = q_ref[0]
    rm = rm_ref[0]
    rmn = rm / jnp.sqrt(jnp.sum(rm * rm, axis=0, keepdims=True))
    mm = jnp.dot(nq.astype(jnp.bfloat16), rmn.astype(jnp.bfloat16),
                 preferred_element_type=jnp.float32)

    iota_l = lax.broadcasted_iota(jnp.int32, (L, NB), 1)
    ri = lax.broadcasted_iota(jnp.int32, (128, 128), 0)
    ci = lax.broadcasted_iota(jnp.int32, (128, 128), 1)
    tril128 = (ci <= ri).astype(jnp.float32)
    cols = []
    for s in range(R):
        ms = mm[:, s * 16:(s + 1) * 16]
        sc = jnp.concatenate([ms, -ms], axis=1)
        mx = jnp.max(sc, axis=1, keepdims=True)
        h = jnp.min(jnp.where(sc == mx, iota_l, NB), axis=1, keepdims=True)
        onehot = (iota_l == h).astype(jnp.float32)
        carry = jnp.zeros((1, NB), jnp.float32)
        parts = []
        for k in range(L // 128):
            blk = onehot[k * 128:(k + 1) * 128]
            cs = jnp.dot(tril128, blk, preferred_element_type=jnp.float32) + carry
            parts.append(cs)
            carry = cs[127:128, :]
        csum = jnp.concatenate(parts, axis=0)
        incl = carry
        for sh in (1, 2, 4, 8, 16):
            incl = incl + jnp.concatenate(
                [jnp.zeros((1, sh), jnp.float32), incl[:, :NB - sh]], axis=1)
        start = incl - carry
        rank_f = (jnp.sum(csum * onehot, axis=1, keepdims=True) - 1.0
                  + jnp.sum(onehot * start, axis=1, keepdims=True))
        rank_ref[0, s] = rank_f.astype(jnp.int32)
        cols.append(h.astype(jnp.float32))
        cols.append(jnp.floor(rank_f / BL))

    cols8 = jnp.concatenate(cols, axis=1)
    eye = _eye64()
    partsT = []
    for k in range(L // 64):
        blk = cols8[k * 64:(k + 1) * 64]
        partsT.append(lax.dot_general(blk, eye, (((0,), (0,)), ((), ())),
                                      preferred_element_type=jnp.float32))
    rows8 = jnp.concatenate(partsT, axis=1)
    meta_ref[0] = rows8.astype(jnp.int32)


def _run_k1(query, rm2d):
    return pl.pallas_call(
        _k1_body,
        grid=(B,),
        in_specs=[
            pl.BlockSpec((1, L, DK), lambda b: (b, 0, 0)),
            pl.BlockSpec((1, DK, R * 16), lambda b: (b, 0, 0)),
        ],
        out_specs=[
            pl.BlockSpec((1, R, L, 1), lambda b: (b, 0, 0, 0)),
            pl.BlockSpec((1, 8, L), lambda b: (b, 0, 0)),
        ],
        out_shape=[
            jax.ShapeDtypeStruct((B, R, L, 1), jnp.int32),
            jax.ShapeDtypeStruct((B, 8, L), jnp.int32),
        ],
    )(query, rm2d)


def _k2_body(qv_hbm, meta_hbm, rank_hbm, qvs_hbm, ms_hbm, qc_hbm,
             rkfull, idxg, qvbuf, min0, min1, min2, min3,
             mo0, mo1, mo2, mo3, qcbuf, sem):
    wid = lax.axis_index("s") * 2 + lax.axis_index("c")
    mins = [min0, min1, min2, min3]
    mouts = [mo0, mo1, mo2, mo3]

    def pair_step(t, _):
        pair = wid * 4 + t
        b = pair // R
        s_ = pair % R
        base = pair * L
        pltpu.sync_copy(rank_hbm.at[pl.ds(base, L)], rkfull)
        pltpu.sync_copy(meta_hbm.at[pl.ds((b * 8 + 2 * s_) * L, L)], mins[0])
        for c in range(1, R):
            sp = (s_ + c) % R
            pltpu.sync_copy(meta_hbm.at[pl.ds((b * 8 + 2 * sp + 1) * L, L)],
                            mins[c])

        def mstep(i, _):
            idx = rkfull[pl.ds(i * 16, 16)]
            idx4 = idx * 4
            for cc in range(4):
                vals = mins[cc][pl.ds(i * 16, 16)]
                plsc.store_scatter(mouts[cc], [idx], vals)
                plsc.store_scatter(qcbuf, [idx4 + cc], vals)
            return 0
        lax.fori_loop(0, L // 16, mstep, 0)
        for cc in range(4):
            pltpu.sync_copy(mouts[cc], ms_hbm.at[pl.ds((pair * 4 + cc) * L, L)])
        pltpu.sync_copy(qcbuf, qc_hbm.at[pl.ds(pair * L * 4, L * 4)])

        for c4 in range(4):
            l0 = c4 * 512
            pltpu.sync_copy(qv_hbm.at[b, pl.ds(l0, 512)], qvbuf)
            for i in range(32):
                row, off = i // 8, (i % 8) * 16
                idxg[row, pl.ds(off, 16)] = rkfull[pl.ds(l0 + i * 16, 16)] + base
            handles = []
            for j in range(4):
                handles.append(pltpu.async_copy(
                    qvbuf.at[pl.ds(j * 128, 128)], qvs_hbm.at[idxg.at[j]], sem))
            for hcp in handles:
                hcp.wait()
        return 0

    lax.fori_loop(0, 4, pair_step, 0)


def _run_k2(qv, meta, rank_flat):
    mesh = plsc.VectorSubcoreMesh(core_axis_name="c", subcore_axis_name="s")
    fn = pl.kernel(
        _k2_body,
        out_type=[
            jax.ShapeDtypeStruct((B * R * L, 2 * DK), jnp.float32),
            jax.ShapeDtypeStruct((B * R * 4 * L,), jnp.int32),
            jax.ShapeDtypeStruct((B * R * L * 4,), jnp.int32),
        ],
        mesh=mesh,
        compiler_params=pltpu.CompilerParams(needs_layout_passes=False),
        scratch_types=(
            [pltpu.VMEM((L,), jnp.int32),
             pltpu.VMEM((4, 128), jnp.int32),
             pltpu.VMEM((512, 2 * DK), jnp.float32)]
            + [pltpu.VMEM((L,), jnp.int32) for _ in range(8)]
            + [pltpu.VMEM((L * 4,), jnp.int32)]
            + [pltpu.SemaphoreType.DMA]
        ),
    )
    return fn(qv, meta, rank_flat)


def _k3_body(qvs_ref, ms_ref, qc_ref, att_ref):
    qvs = qvs_ref[0, 0]
    qs = qvs[:, 0:DK]
    vs = qvs[:, DK:2 * DK]
    mrows = ms_ref[0, 0].astype(jnp.float32)

    kn = qs / jnp.maximum(jnp.sqrt(jnp.sum(qs * qs, axis=1, keepdims=True)), 1e-12)
    kall = jnp.concatenate([kn[L - BL:], kn], axis=0)
    vall = jnp.concatenate([vs[L - BL:], vs], axis=0)
    mcat = jnp.concatenate([mrows[:, L - BL:], mrows], axis=1)
    qcolf = qc_ref[0, 0].astype(jnp.float32)

    inv_sqrt = 1.0 / math.sqrt(DK)
    CH = 256
    ri = lax.broadcasted_iota(jnp.int32, (CH, 2 * BL), 0)
    ci = lax.broadcasted_iota(jnp.int32, (CH, 2 * BL), 1)
    i64 = ri % BL
    later = jnp.logical_and(ci >= BL, ci - BL > i64)
    selfm = (ci - BL == i64)
    wrap0 = jnp.logical_or(later, jnp.logical_and(ri < BL, ci < BL))

    for u in range(L // CH):
        r0 = u * CH
        qk_parts = []
        for m in range(4):
            n = u * 4 + m
            qk_parts.append(lax.dot_general(
                qs[n * BL:(n + 1) * BL], kall[n * BL:n * BL + 2 * BL],
                (((1,), (1,)), ((), ())),
                preferred_element_type=jnp.float32))
        qk = jnp.concatenate(qk_parts, axis=0) * inv_sqrt
        kband = []
        for c in range(4):
            kband.append(jnp.concatenate(
                [jnp.broadcast_to(mcat[c:c + 1, m * BL:m * BL + 2 * BL], (BL, 2 * BL))
                 for m in range(u * 4, u * 4 + 4)], axis=0))
        qcol = qcolf[r0:r0 + CH]
        qk = jnp.where(qcol[:, 0:1] != kband[0], NEG_BIG, qk)
        qk = jnp.where(wrap0 if u == 0 else later, NEG_BIG, qk)
        qk = jnp.where(selfm, NEG_SELF, qk)
        ck = jnp.ones((CH, 2 * BL), jnp.float32)
        for c in range(1, 4):
            bqc = qcol[:, c:c + 1]
            prv = jnp.where(bqc == 0.0, float(NB - 1), bqc - 1.0)
            ck = (ck + (kband[c] == bqc).astype(jnp.float32)
                  + (kband[c] == prv).astype(jnp.float32))
        m_ = jnp.max(qk, axis=1, keepdims=True)
        p = jnp.exp(qk - m_)
        ssum = jnp.sum(p, axis=1, keepdims=True)
        sm = p / (ssum * ck)
        att_ref[0, 0, r0:r0 + CH, DK:DK + 1] = m_ + jnp.log(ssum)
        att_ref[0, 0, r0:r0 + CH, DK + 1:2 * DK] = jnp.zeros(
            (CH, DK - 1), jnp.float32)
        for m in range(4):
            n = u * 4 + m
            att_ref[0, 0, n * BL:(n + 1) * BL, 0:DK] = jnp.dot(
                sm[m * BL:(m + 1) * BL], vall[n * BL:n * BL + 2 * BL],
                preferred_element_type=jnp.float32)


def _run_k3(qvs, ms, qc):
    return pl.pallas_call(
        _k3_body,
        grid=(B, R),
        in_specs=[
            pl.BlockSpec((1, 1, L, 2 * DK), lambda b, s: (b, s, 0, 0)),
            pl.BlockSpec((1, 1, 4, L), lambda b, s: (b, s, 0, 0)),
            pl.BlockSpec((1, 1, L, 4), lambda b, s: (b, s, 0, 0)),
        ],
        out_specs=pl.BlockSpec((1, 1, L, 2 * DK), lambda b, s: (b, s, 0, 0)),
        out_shape=jax.ShapeDtypeStruct((B, R, L, 2 * DK), jnp.float32),
    )(qvs, ms, qc)


def _k4_body(att_hbm, rank_hbm, atto_hbm, rkfull, idxg, abuf, sem):
    wid = lax.axis_index("s") * 2 + lax.axis_index("c")

    def step(it, _):
        t = it // 4
        c4 = it % 4
        pair = wid * 4 + t
        b = pair // R
        s_ = pair % R
        l0 = c4 * 512
        base = pair * L
        pltpu.sync_copy(rank_hbm.at[pl.ds(base + l0, 512)], rkfull)
        for i in range(32):
            row, off = i // 8, (i % 8) * 16
            idxg[row, pl.ds(off, 16)] = rkfull[pl.ds(i * 16, 16)] + base
        handles = []
        for j in range(4):
            handles.append(pltpu.async_copy(
                att_hbm.at[idxg.at[j]], abuf.at[pl.ds(j * 128, 128)], sem))
        for hcp in handles:
            hcp.wait()
        pltpu.sync_copy(abuf, atto_hbm.at[b, s_, pl.ds(l0, 512)])
        return 0

    lax.fori_loop(0, 16, step, 0)


def _run_k4(att_flat, rank_flat):
    mesh = plsc.VectorSubcoreMesh(core_axis_name="c", subcore_axis_name="s")
    fn = pl.kernel(
        _k4_body,
        out_type=jax.ShapeDtypeStruct((B, R, L, 2 * DK), jnp.float32),
        mesh=mesh,
        compiler_params=pltpu.CompilerParams(needs_layout_passes=False),
        scratch_types=[
            pltpu.VMEM((512,), jnp.int32),
            pltpu.VMEM((4, 128), jnp.int32),
            pltpu.VMEM((512, 2 * DK), jnp.float32),
            pltpu.SemaphoreType.DMA,
        ],
    )
    return fn(att_flat, rank_flat)


def _k5_body(att_ref, out_ref):
    acc = jnp.zeros((L, DK), jnp.float32)
    for s in range(R):
        col = att_ref[0, s, :, DK:DK + 1]
        m = jnp.max(col, axis=0, keepdims=True)
        e = jnp.exp(col - m)
        w = e / jnp.sum(e, axis=0, keepdims=True)
        acc = acc + w * att_ref[0, s, :, 0:DK]
    out_ref[0] = acc


def _run_k5(att_o):
    return pl.pallas_call(
        _k5_body,
        grid=(B,),
        in_specs=[pl.BlockSpec((1, R, L, 2 * DK), lambda b: (b, 0, 0, 0))],
        out_specs=pl.BlockSpec((1, L, DK), lambda b: (b, 0, 0)),
        out_shape=jax.ShapeDtypeStruct((B, L, DK), jnp.float32),
    )(att_o)


def kernel(query, value, seed, rand_matrix):
    rm2d = rand_matrix.reshape(B, DK, R * 16)
    qv = jnp.concatenate([query, value], axis=-1)
    nq = query / jnp.maximum(
        jnp.sqrt(jnp.sum(query * query, axis=-1, keepdims=True)), 1e-12)
    rank, meta = _run_k1(nq, rm2d)
    rank_flat = rank.reshape(B * R * L)
    qvs, ms, qc = _run_k2(qv, meta.reshape(B * 8 * L), rank_flat)
    att = _run_k3(qvs.reshape(B, R, L, 2 * DK), ms.reshape(B, R, 4, L),
                  qc.reshape(B, R, L, 4))
    att_o = _run_k4(att.reshape(B * R * L, 2 * DK), rank_flat)
    return _run_k5(att_o)


# speedup vs baseline: 1.1215x; 1.0577x over previous
import math

import jax
import jax.numpy as jnp
from jax import lax
from jax.experimental import pallas as pl
from jax.experimental.pallas import tpu as pltpu
from jax.experimental.pallas import tpu_sc as plsc

B, L, DK, R, BL, NB = 32, 2048, 64, 4, 64, 32
NEG_BIG = -1e9
NEG_SELF = -1e5


def _eye64():
    i = lax.broadcasted_iota(jnp.int32, (64, 64), 0)
    j = lax.broadcasted_iota(jnp.int32, (64, 64), 1)
    return (i == j).astype(jnp.float32)


def _k1_body(q_ref, rm_ref, rank_ref, meta_ref):
    nq = q_ref[0]
    rm = rm_ref[0]
    rmn = rm / jnp.sqrt(jnp.sum(rm * rm, axis=0, keepdims=True))
    mm = jnp.dot(nq.astype(jnp.bfloat16), rmn.astype(jnp.bfloat16),
                 preferred_element_type=jnp.float32)

    iota_l = lax.broadcasted_iota(jnp.int32, (L, NB), 1)
    ri = lax.broadcasted_iota(jnp.int32, (128, 128), 0)
    ci = lax.broadcasted_iota(jnp.int32, (128, 128), 1)
    tril128 = (ci <= ri).astype(jnp.float32)
    cols = []
    for s in range(R):
        ms = mm[:, s * 16:(s + 1) * 16]
        sc = jnp.concatenate([ms, -ms], axis=1)
        mx = jnp.max(sc, axis=1, keepdims=True)
        h = jnp.min(jnp.where(sc == mx, iota_l, NB), axis=1, keepdims=True)
        onehot = (iota_l == h).astype(jnp.float32)
        carry = jnp.zeros((1, NB), jnp.float32)
        parts = []
        for k in range(L // 128):
            blk = onehot[k * 128:(k + 1) * 128]
            cs = jnp.dot(tril128, blk, preferred_element_type=jnp.float32) + carry
            parts.append(cs)
            carry = cs[127:128, :]
        csum = jnp.concatenate(parts, axis=0)
        incl = carry
        for sh in (1, 2, 4, 8, 16):
            incl = incl + jnp.concatenate(
                [jnp.zeros((1, sh), jnp.float32), incl[:, :NB - sh]], axis=1)
        start = incl - carry
        rank_f = (jnp.sum(csum * onehot, axis=1, keepdims=True) - 1.0
                  + jnp.sum(onehot * start, axis=1, keepdims=True))
        rank_ref[0, s] = rank_f.astype(jnp.int32)
        cols.append(h.astype(jnp.float32))
        cols.append(jnp.floor(rank_f / BL))

    cols8 = jnp.concatenate(cols, axis=1)
    eye = _eye64()
    partsT = []
    for k in range(L // 64):
        blk = cols8[k * 64:(k + 1) * 64]
        partsT.append(lax.dot_general(blk, eye, (((0,), (0,)), ((), ())),
                                      preferred_element_type=jnp.float32))
    rows8 = jnp.concatenate(partsT, axis=1)
    meta_ref[0] = rows8.astype(jnp.int32)


def _run_k1(query, rm2d):
    return pl.pallas_call(
        _k1_body,
        grid=(B,),
        in_specs=[
            pl.BlockSpec((1, L, DK), lambda b: (b, 0, 0)),
            pl.BlockSpec((1, DK, R * 16), lambda b: (b, 0, 0)),
        ],
        out_specs=[
            pl.BlockSpec((1, R, L, 1), lambda b: (b, 0, 0, 0)),
            pl.BlockSpec((1, 8, L), lambda b: (b, 0, 0)),
        ],
        out_shape=[
            jax.ShapeDtypeStruct((B, R, L, 1), jnp.int32),
            jax.ShapeDtypeStruct((B, 8, L), jnp.int32),
        ],
    )(query, rm2d)


def _make_k2_body(soff):
    def _k2_body(qv_hbm, meta_hbm, rank_hbm, qvs_hbm, ms_hbm, qc_hbm,
                 rkfull, idxg, qvbuf, min0, min1, min2, min3,
                 mo0, mo1, mo2, mo3, qcbuf, sem):
        wid = lax.axis_index("s") * 2 + lax.axis_index("c")
        mins = [min0, min1, min2, min3]
        mouts = [mo0, mo1, mo2, mo3]

        def pair_step(t, _):
            k = wid * 2 + t
            b = k // 2
            s_ = soff + k % 2
            base = k * L
            pltpu.sync_copy(rank_hbm.at[pl.ds((b * R + s_) * L, L)], rkfull)
            pltpu.sync_copy(meta_hbm.at[pl.ds((b * 8 + 2 * s_) * L, L)], mins[0])
            for c in range(1, R):
                sp = (s_ + c) % R
                pltpu.sync_copy(meta_hbm.at[pl.ds((b * 8 + 2 * sp + 1) * L, L)],
                                mins[c])

            def mstep(i, _):
                idx = rkfull[pl.ds(i * 16, 16)]
                idx4 = idx * 4
                for cc in range(4):
                    vals = mins[cc][pl.ds(i * 16, 16)]
                    plsc.store_scatter(mouts[cc], [idx], vals)
                    plsc.store_scatter(qcbuf, [idx4 + cc], vals)
                return 0
            lax.fori_loop(0, L // 16, mstep, 0)
            for cc in range(4):
                pltpu.sync_copy(mouts[cc], ms_hbm.at[pl.ds((k * 4 + cc) * L, L)])
            pltpu.sync_copy(qcbuf, qc_hbm.at[pl.ds(k * L * 4, L * 4)])

            for c4 in range(4):
                l0 = c4 * 512
                pltpu.sync_copy(qv_hbm.at[b, pl.ds(l0, 512)], qvbuf)
                for i in range(32):
                    row, off = i // 8, (i % 8) * 16
                    idxg[row, pl.ds(off, 16)] = rkfull[pl.ds(l0 + i * 16, 16)] + base
                handles = []
                for j in range(4):
                    handles.append(pltpu.async_copy(
                        qvbuf.at[pl.ds(j * 128, 128)], qvs_hbm.at[idxg.at[j]], sem))
                for hcp in handles:
                    hcp.wait()
            return 0

        lax.fori_loop(0, 2, pair_step, 0)
    return _k2_body


def _run_k2(qv, meta, rank_flat, soff):
    mesh = plsc.VectorSubcoreMesh(core_axis_name="c", subcore_axis_name="s")
    fn = pl.kernel(
        _make_k2_body(soff),
        out_type=[
            jax.ShapeDtypeStruct((B * 2 * L, 2 * DK), jnp.float32),
            jax.ShapeDtypeStruct((B * 2 * 4 * L,), jnp.int32),
            jax.ShapeDtypeStruct((B * 2 * L * 4,), jnp.int32),
        ],
        mesh=mesh,
        compiler_params=pltpu.CompilerParams(needs_layout_passes=False),
        scratch_types=(
            [pltpu.VMEM((L,), jnp.int32),
             pltpu.VMEM((4, 128), jnp.int32),
             pltpu.VMEM((512, 2 * DK), jnp.float32)]
            + [pltpu.VMEM((L,), jnp.int32) for _ in range(8)]
            + [pltpu.VMEM((L * 4,), jnp.int32)]
            + [pltpu.SemaphoreType.DMA]
        ),
    )
    return fn(qv, meta, rank_flat)


def _k3_body(qvs_ref, ms_ref, qc_ref, att_ref):
    qvs = qvs_ref[0, 0]
    qs = qvs[:, 0:DK]
    vs = qvs[:, DK:2 * DK]
    mrows = ms_ref[0, 0].astype(jnp.float32)

    kn = qs / jnp.maximum(jnp.sqrt(jnp.sum(qs * qs, axis=1, keepdims=True)), 1e-12)
    kall = jnp.concatenate([kn[L - BL:], kn], axis=0)
    vall = jnp.concatenate([vs[L - BL:], vs], axis=0)
    mcat = jnp.concatenate([mrows[:, L - BL:], mrows], axis=1)
    qcolf = qc_ref[0, 0].astype(jnp.float32)

    inv_sqrt = 1.0 / math.sqrt(DK)
    CH = 256
    ri = lax.broadcasted_iota(jnp.int32, (CH, 2 * BL), 0)
    ci = lax.broadcasted_iota(jnp.int32, (CH, 2 * BL), 1)
    i64 = ri % BL
    later = jnp.logical_and(ci >= BL, ci - BL > i64)
    selfm = (ci - BL == i64)
    wrap0 = jnp.logical_or(later, jnp.logical_and(ri < BL, ci < BL))

    for u in range(L // CH):
        r0 = u * CH
        qk_parts = []
        for m in range(4):
            n = u * 4 + m
            qk_parts.append(lax.dot_general(
                qs[n * BL:(n + 1) * BL], kall[n * BL:n * BL + 2 * BL],
                (((1,), (1,)), ((), ())),
                preferred_element_type=jnp.float32))
        qk = jnp.concatenate(qk_parts, axis=0) * inv_sqrt
        kband = []
        for c in range(4):
            kband.append(jnp.concatenate(
                [jnp.broadcast_to(mcat[c:c + 1, m * BL:m * BL + 2 * BL], (BL, 2 * BL))
                 for m in range(u * 4, u * 4 + 4)], axis=0))
        qcol = qcolf[r0:r0 + CH]
        qk = jnp.where(qcol[:, 0:1] != kband[0], NEG_BIG, qk)
        qk = jnp.where(wrap0 if u == 0 else later, NEG_BIG, qk)
        qk = jnp.where(selfm, NEG_SELF, qk)
        ck = jnp.ones((CH, 2 * BL), jnp.float32)
        for c in range(1, 4):
            bqc = qcol[:, c:c + 1]
            prv = jnp.where(bqc == 0.0, float(NB - 1), bqc - 1.0)
            ck = (ck + (kband[c] == bqc).astype(jnp.float32)
                  + (kband[c] == prv).astype(jnp.float32))
        m_ = jnp.max(qk, axis=1, keepdims=True)
        p = jnp.exp(qk - m_)
        ssum = jnp.sum(p, axis=1, keepdims=True)
        sm = p / (ssum * ck)
        att_ref[0, 0, r0:r0 + CH, DK:DK + 1] = m_ + jnp.log(ssum)
        att_ref[0, 0, r0:r0 + CH, DK + 1:2 * DK] = jnp.zeros(
            (CH, DK - 1), jnp.float32)
        for m in range(4):
            n = u * 4 + m
            att_ref[0, 0, n * BL:(n + 1) * BL, 0:DK] = jnp.dot(
                sm[m * BL:(m + 1) * BL], vall[n * BL:n * BL + 2 * BL],
                preferred_element_type=jnp.float32)


def _run_k3(qvs, ms, qc):
    return pl.pallas_call(
        _k3_body,
        grid=(B, 2),
        in_specs=[
            pl.BlockSpec((1, 1, L, 2 * DK), lambda b, s: (b, s, 0, 0)),
            pl.BlockSpec((1, 1, 4, L), lambda b, s: (b, s, 0, 0)),
            pl.BlockSpec((1, 1, L, 4), lambda b, s: (b, s, 0, 0)),
        ],
        out_specs=pl.BlockSpec((1, 1, L, 2 * DK), lambda b, s: (b, s, 0, 0)),
        out_shape=jax.ShapeDtypeStruct((B, 2, L, 2 * DK), jnp.float32),
    )(qvs, ms, qc)


def _make_k4_body(soff):
    def _k4_body(att_hbm, rank_hbm, atto_hbm, rkfull, idxg, abuf, sem):
        wid = lax.axis_index("s") * 2 + lax.axis_index("c")

        def step2(it, _):
            t = it // 4
            c4 = it % 4
            k = wid * 2 + t
            b = k // 2
            s_ = soff + k % 2
            l0 = c4 * 512
            base = k * L
            pltpu.sync_copy(rank_hbm.at[pl.ds((b * R + s_) * L + l0, 512)], rkfull)
            for i in range(32):
                row, off = i // 8, (i % 8) * 16
                idxg[row, pl.ds(off, 16)] = rkfull[pl.ds(i * 16, 16)] + base
            handles = []
            for j in range(4):
                handles.append(pltpu.async_copy(
                    att_hbm.at[idxg.at[j]], abuf.at[pl.ds(j * 128, 128)], sem))
            for hcp in handles:
                hcp.wait()
            pltpu.sync_copy(abuf, atto_hbm.at[b, k % 2, pl.ds(l0, 512)])
            return 0

        lax.fori_loop(0, 8, step2, 0)
    return _k4_body


def _run_k4(att_flat, rank_flat, soff):
    mesh = plsc.VectorSubcoreMesh(core_axis_name="c", subcore_axis_name="s")
    fn = pl.kernel(
        _make_k4_body(soff),
        out_type=jax.ShapeDtypeStruct((B, 2, L, 2 * DK), jnp.float32),
        mesh=mesh,
        compiler_params=pltpu.CompilerParams(needs_layout_passes=False),
        scratch_types=[
            pltpu.VMEM((512,), jnp.int32),
            pltpu.VMEM((4, 128), jnp.int32),
            pltpu.VMEM((512, 2 * DK), jnp.float32),
            pltpu.SemaphoreType.DMA,
        ],
    )
    return fn(att_flat, rank_flat)


def _k5_body(atta_ref, attb_ref, out_ref):
    acc = jnp.zeros((L, DK), jnp.float32)
    for ref in (atta_ref, attb_ref):
        for s in range(2):
            col = ref[0, s, :, DK:DK + 1]
            m = jnp.max(col, axis=0, keepdims=True)
            e = jnp.exp(col - m)
            w = e / jnp.sum(e, axis=0, keepdims=True)
            acc = acc + w * ref[0, s, :, 0:DK]
    out_ref[0] = acc


def _run_k5(att_a, att_b):
    return pl.pallas_call(
        _k5_body,
        grid=(B,),
        in_specs=[pl.BlockSpec((1, 2, L, 2 * DK), lambda b: (b, 0, 0, 0)),
                  pl.BlockSpec((1, 2, L, 2 * DK), lambda b: (b, 0, 0, 0))],
        out_specs=pl.BlockSpec((1, L, DK), lambda b: (b, 0, 0)),
        out_shape=jax.ShapeDtypeStruct((B, L, DK), jnp.float32),
    )(att_a, att_b)


def kernel(query, value, seed, rand_matrix):
    rm2d = rand_matrix.reshape(B, DK, R * 16)
    qv = jnp.concatenate([query, value], axis=-1)
    nq = query / jnp.maximum(
        jnp.sqrt(jnp.sum(query * query, axis=-1, keepdims=True)), 1e-12)
    rank, meta = _run_k1(nq, rm2d)
    rank_flat = rank.reshape(B * R * L)
    meta_flat = meta.reshape(B * 8 * L)
    halves = []
    for soff in (0, 2):
        qvs, ms, qc = _run_k2(qv, meta_flat, rank_flat, soff)
        att = _run_k3(qvs.reshape(B, 2, L, 2 * DK), ms.reshape(B, 2, 4, L),
                      qc.reshape(B, 2, L, 4))
        halves.append(_run_k4(att.reshape(B * 2 * L, 2 * DK), rank_flat, soff))
    return _run_k5(halves[0], halves[1])


# speedup vs baseline: 1.1596x; 1.0340x over previous
import math

import jax
import jax.numpy as jnp
from jax import lax
from jax.experimental import pallas as pl
from jax.experimental.pallas import tpu as pltpu
from jax.experimental.pallas import tpu_sc as plsc

B, L, DK, R, BL, NB = 32, 2048, 64, 4, 64, 32
NEG_BIG = -1e9
NEG_SELF = -1e5


def _eye64():
    i = lax.broadcasted_iota(jnp.int32, (64, 64), 0)
    j = lax.broadcasted_iota(jnp.int32, (64, 64), 1)
    return (i == j).astype(jnp.float32)


def _k1_body(q_ref, rm_ref, rank_ref, meta_ref):
    nq = q_ref[0]
    rm = rm_ref[0]
    rmn = rm / jnp.sqrt(jnp.sum(rm * rm, axis=0, keepdims=True))
    mm = jnp.dot(nq.astype(jnp.bfloat16), rmn.astype(jnp.bfloat16),
                 preferred_element_type=jnp.float32)

    iota_l = lax.broadcasted_iota(jnp.int32, (L, NB), 1)
    ri = lax.broadcasted_iota(jnp.int32, (128, 128), 0)
    ci = lax.broadcasted_iota(jnp.int32, (128, 128), 1)
    tril128 = (ci <= ri).astype(jnp.float32)
    r32 = lax.broadcasted_iota(jnp.int32, (NB, NB), 0)
    c32 = lax.broadcasted_iota(jnp.int32, (NB, NB), 1)
    upper32 = (r32 < c32).astype(jnp.float32)
    cols = []
    for s in range(R):
        ms = mm[:, s * 16:(s + 1) * 16]
        sc = jnp.concatenate([ms, -ms], axis=1)
        mx = jnp.max(sc, axis=1, keepdims=True)
        h = jnp.min(jnp.where(sc == mx, iota_l, NB), axis=1, keepdims=True)
        onehot = (iota_l == h).astype(jnp.float32)
        carry = jnp.zeros((1, NB), jnp.float32)
        parts = []
        for k in range(L // 128):
            blk = onehot[k * 128:(k + 1) * 128]
            cs = jnp.dot(tril128, blk, preferred_element_type=jnp.float32) + carry
            parts.append(cs)
            carry = cs[127:128, :]
        csum = jnp.concatenate(parts, axis=0)
        hi = jnp.floor(carry * (1.0 / 256.0))
        lo = carry - hi * 256.0
        start = (jnp.dot(hi, upper32, preferred_element_type=jnp.float32)
                 * 256.0
                 + jnp.dot(lo, upper32, preferred_element_type=jnp.float32))
        rank_f = (jnp.sum(csum * onehot, axis=1, keepdims=True) - 1.0
                  + jnp.sum(onehot * start, axis=1, keepdims=True))
        rank_ref[0, s] = rank_f.astype(jnp.int32)
        cols.append(h.astype(jnp.float32))
        cols.append(jnp.floor(rank_f / BL))

    cols8 = jnp.concatenate(cols, axis=1)
    eye = _eye64()
    partsT = []
    for k in range(L // 64):
        blk = cols8[k * 64:(k + 1) * 64]
        partsT.append(lax.dot_general(blk, eye, (((0,), (0,)), ((), ())),
                                      preferred_element_type=jnp.float32))
    rows8 = jnp.concatenate(partsT, axis=1)
    meta_ref[0] = rows8.astype(jnp.int32)


def _run_k1(query, rm2d):
    return pl.pallas_call(
        _k1_body,
        grid=(B,),
        in_specs=[
            pl.BlockSpec((1, L, DK), lambda b: (b, 0, 0)),
            pl.BlockSpec((1, DK, R * 16), lambda b: (b, 0, 0)),
        ],
        out_specs=[
            pl.BlockSpec((1, R, L, 1), lambda b: (b, 0, 0, 0)),
            pl.BlockSpec((1, 8, L), lambda b: (b, 0, 0)),
        ],
        out_shape=[
            jax.ShapeDtypeStruct((B, R, L, 1), jnp.int32),
            jax.ShapeDtypeStruct((B, 8, L), jnp.int32),
        ],
    )(query, rm2d)


def _make_k2_body(soff):
    def _k2_body(qv_hbm, meta_hbm, rank_hbm, qvs_hbm, ms_hbm, qc_hbm,
                 rkfull, idxg, qvbuf, min0, min1, min2, min3,
                 mo0, mo1, mo2, mo3, qcbuf, sem):
        wid = lax.axis_index("s") * 2 + lax.axis_index("c")
        mins = [min0, min1, min2, min3]
        mouts = [mo0, mo1, mo2, mo3]

        def pair_step(t, _):
            k = wid * 2 + t
            b = k // 2
            s_ = soff + k % 2
            base = k * L
            pltpu.sync_copy(rank_hbm.at[pl.ds((b * R + s_) * L, L)], rkfull)
            pltpu.sync_copy(meta_hbm.at[pl.ds((b * 8 + 2 * s_) * L, L)], mins[0])
            for c in range(1, R):
                sp = (s_ + c) % R
                pltpu.sync_copy(meta_hbm.at[pl.ds((b * 8 + 2 * sp + 1) * L, L)],
                                mins[c])

            def mstep(i, _):
                idx = rkfull[pl.ds(i * 16, 16)]
                idx4 = idx * 4
                for cc in range(4):
                    vals = mins[cc][pl.ds(i * 16, 16)]
                    plsc.store_scatter(mouts[cc], [idx], vals)
                    plsc.store_scatter(qcbuf, [idx4 + cc], vals)
                return 0
            lax.fori_loop(0, L // 16, mstep, 0)
            for cc in range(4):
                pltpu.sync_copy(mouts[cc], ms_hbm.at[pl.ds((k * 4 + cc) * L, L)])
            pltpu.sync_copy(qcbuf, qc_hbm.at[pl.ds(k * L * 4, L * 4)])

            for c4 in range(4):
                l0 = c4 * 512
                pltpu.sync_copy(qv_hbm.at[b, pl.ds(l0, 512)], qvbuf)
                for i in range(32):
                    row, off = i // 8, (i % 8) * 16
                    idxg[row, pl.ds(off, 16)] = rkfull[pl.ds(l0 + i * 16, 16)] + base
                handles = []
                for j in range(4):
                    handles.append(pltpu.async_copy(
                        qvbuf.at[pl.ds(j * 128, 128)], qvs_hbm.at[idxg.at[j]], sem))
                for hcp in handles:
                    hcp.wait()
            return 0

        lax.fori_loop(0, 2, pair_step, 0)
    return _k2_body


def _run_k2(qv, meta, rank_flat, soff):
    mesh = plsc.VectorSubcoreMesh(core_axis_name="c", subcore_axis_name="s")
    fn = pl.kernel(
        _make_k2_body(soff),
        out_type=[
            jax.ShapeDtypeStruct((B * 2 * L, 2 * DK), jnp.float32),
            jax.ShapeDtypeStruct((B * 2 * 4 * L,), jnp.int32),
            jax.ShapeDtypeStruct((B * 2 * L * 4,), jnp.int32),
        ],
        mesh=mesh,
        compiler_params=pltpu.CompilerParams(needs_layout_passes=False),
        scratch_types=(
            [pltpu.VMEM((L,), jnp.int32),
             pltpu.VMEM((4, 128), jnp.int32),
             pltpu.VMEM((512, 2 * DK), jnp.float32)]
            + [pltpu.VMEM((L,), jnp.int32) for _ in range(8)]
            + [pltpu.VMEM((L * 4,), jnp.int32)]
            + [pltpu.SemaphoreType.DMA]
        ),
    )
    return fn(qv, meta, rank_flat)


def _k3_body(qvs_ref, ms_ref, qc_ref, att_ref):
    qvs = qvs_ref[0, 0]
    qs = qvs[:, 0:DK]
    vs = qvs[:, DK:2 * DK]
    mrows = ms_ref[0, 0].astype(jnp.float32)

    kn = qs / jnp.maximum(jnp.sqrt(jnp.sum(qs * qs, axis=1, keepdims=True)), 1e-12)
    kall = jnp.concatenate([kn[L - BL:], kn], axis=0)
    vall = jnp.concatenate([vs[L - BL:], vs], axis=0)
    mcat = jnp.concatenate([mrows[:, L - BL:], mrows], axis=1)
    qcolf = qc_ref[0, 0].astype(jnp.float32)

    inv_sqrt = 1.0 / math.sqrt(DK)
    CH = 256
    ri = lax.broadcasted_iota(jnp.int32, (CH, 2 * BL), 0)
    ci = lax.broadcasted_iota(jnp.int32, (CH, 2 * BL), 1)
    i64 = ri % BL
    later = jnp.logical_and(ci >= BL, ci - BL > i64)
    selfm = (ci - BL == i64)
    wrap0 = jnp.logical_or(later, jnp.logical_and(ri < BL, ci < BL))

    for u in range(L // CH):
        r0 = u * CH
        qk_parts = []
        for m in range(4):
            n = u * 4 + m
            qk_parts.append(lax.dot_general(
                qs[n * BL:(n + 1) * BL], kall[n * BL:n * BL + 2 * BL],
                (((1,), (1,)), ((), ())),
                preferred_element_type=jnp.float32))
        qk = jnp.concatenate(qk_parts, axis=0) * inv_sqrt
        kband = []
        for c in range(4):
            kband.append(jnp.concatenate(
                [jnp.broadcast_to(mcat[c:c + 1, m * BL:m * BL + 2 * BL], (BL, 2 * BL))
                 for m in range(u * 4, u * 4 + 4)], axis=0))
        qcol = qcolf[r0:r0 + CH]
        qk = jnp.where(qcol[:, 0:1] != kband[0], NEG_BIG, qk)
        qk = jnp.where(wrap0 if u == 0 else later, NEG_BIG, qk)
        qk = jnp.where(selfm, NEG_SELF, qk)
        ck = jnp.ones((CH, 2 * BL), jnp.float32)
        for c in range(1, 4):
            bqc = qcol[:, c:c + 1]
            prv = jnp.where(bqc == 0.0, float(NB - 1), bqc - 1.0)
            ck = (ck + (kband[c] == bqc).astype(jnp.float32)
                  + (kband[c] == prv).astype(jnp.float32))
        m_ = jnp.max(qk, axis=1, keepdims=True)
        p = jnp.exp(qk - m_)
        ssum = jnp.sum(p, axis=1, keepdims=True)
        sm = p / (ssum * ck)
        att_ref[0, 0, r0:r0 + CH, DK:DK + 1] = m_ + jnp.log(ssum)
        att_ref[0, 0, r0:r0 + CH, DK + 1:2 * DK] = jnp.zeros(
            (CH, DK - 1), jnp.float32)
        for m in range(4):
            n = u * 4 + m
            att_ref[0, 0, n * BL:(n + 1) * BL, 0:DK] = jnp.dot(
                sm[m * BL:(m + 1) * BL], vall[n * BL:n * BL + 2 * BL],
                preferred_element_type=jnp.float32)


def _run_k3(qvs, ms, qc):
    return pl.pallas_call(
        _k3_body,
        grid=(B, 2),
        in_specs=[
            pl.BlockSpec((1, 1, L, 2 * DK), lambda b, s: (b, s, 0, 0)),
            pl.BlockSpec((1, 1, 4, L), lambda b, s: (b, s, 0, 0)),
            pl.BlockSpec((1, 1, L, 4), lambda b, s: (b, s, 0, 0)),
        ],
        out_specs=pl.BlockSpec((1, 1, L, 2 * DK), lambda b, s: (b, s, 0, 0)),
        out_shape=jax.ShapeDtypeStruct((B, 2, L, 2 * DK), jnp.float32),
    )(qvs, ms, qc)


def _make_k4_body(soff):
    def _k4_body(att_hbm, rank_hbm, atto_hbm, rkfull, idxg, abuf, sem):
        wid = lax.axis_index("s") * 2 + lax.axis_index("c")

        def step2(it, _):
            t = it // 4
            c4 = it % 4
            k = wid * 2 + t
            b = k // 2
            s_ = soff + k % 2
            l0 = c4 * 512
            base = k * L
            pltpu.sync_copy(rank_hbm.at[pl.ds((b * R + s_) * L + l0, 512)], rkfull)
            for i in range(32):
                row, off = i // 8, (i % 8) * 16
                idxg[row, pl.ds(off, 16)] = rkfull[pl.ds(i * 16, 16)] + base
            handles = []
            for j in range(4):
                handles.append(pltpu.async_copy(
                    att_hbm.at[idxg.at[j]], abuf.at[pl.ds(j * 128, 128)], sem))
            for hcp in handles:
                hcp.wait()
            pltpu.sync_copy(abuf, atto_hbm.at[b, k % 2, pl.ds(l0, 512)])
            return 0

        lax.fori_loop(0, 8, step2, 0)
    return _k4_body


def _run_k4(att_flat, rank_flat, soff):
    mesh = plsc.VectorSubcoreMesh(core_axis_name="c", subcore_axis_name="s")
    fn = pl.kernel(
        _make_k4_body(soff),
        out_type=jax.ShapeDtypeStruct((B, 2, L, 2 * DK), jnp.float32),
        mesh=mesh,
        compiler_params=pltpu.CompilerParams(needs_layout_passes=False),
        scratch_types=[
            pltpu.VMEM((512,), jnp.int32),
            pltpu.VMEM((4, 128), jnp.int32),
            pltpu.VMEM((512, 2 * DK), jnp.float32),
            pltpu.SemaphoreType.DMA,
        ],
    )
    return fn(att_flat, rank_flat)


def _k5_body(atta_ref, attb_ref, out_ref):
    acc = jnp.zeros((L, DK), jnp.float32)
    for ref in (atta_ref, attb_ref):
        for s in range(2):
            col = ref[0, s, :, DK:DK + 1]
            m = jnp.max(col, axis=0, keepdims=True)
            e = jnp.exp(col - m)
            w = e / jnp.sum(e, axis=0, keepdims=True)
            acc = acc + w * ref[0, s, :, 0:DK]
    out_ref[0] = acc


def _run_k5(att_a, att_b):
    return pl.pallas_call(
        _k5_body,
        grid=(B,),
        in_specs=[pl.BlockSpec((1, 2, L, 2 * DK), lambda b: (b, 0, 0, 0)),
                  pl.BlockSpec((1, 2, L, 2 * DK), lambda b: (b, 0, 0, 0))],
        out_specs=pl.BlockSpec((1, L, DK), lambda b: (b, 0, 0)),
        out_shape=jax.ShapeDtypeStruct((B, L, DK), jnp.float32),
    )(att_a, att_b)


def kernel(query, value, seed, rand_matrix):
    rm2d = rand_matrix.reshape(B, DK, R * 16)
    qv = jnp.concatenate([query, value], axis=-1)
    nq = query / jnp.maximum(
        jnp.sqrt(jnp.sum(query * query, axis=-1, keepdims=True)), 1e-12)
    rank, meta = _run_k1(nq, rm2d)
    rank_flat = rank.reshape(B * R * L)
    meta_flat = meta.reshape(B * 8 * L)
    halves = []
    for soff in (0, 2):
        qvs, ms, qc = _run_k2(qv, meta_flat, rank_flat, soff)
        att = _run_k3(qvs.reshape(B, 2, L, 2 * DK), ms.reshape(B, 2, 4, L),
                      qc.reshape(B, 2, L, 4))
        halves.append(_run_k4(att.reshape(B * 2 * L, 2 * DK), rank_flat, soff))
    return _run_k5(halves[0], halves[1])


# speedup vs baseline: 1.1679x; 1.0071x over previous
import math

import jax
import jax.numpy as jnp
from jax import lax
from jax.experimental import pallas as pl
from jax.experimental.pallas import tpu as pltpu
from jax.experimental.pallas import tpu_sc as plsc

B, L, DK, R, BL, NB = 32, 2048, 64, 4, 64, 32
NEG_BIG = -1e9
NEG_SELF = -1e5


def _eye64():
    i = lax.broadcasted_iota(jnp.int32, (64, 64), 0)
    j = lax.broadcasted_iota(jnp.int32, (64, 64), 1)
    return (i == j).astype(jnp.float32)


def _k1_body(q_ref, rm_ref, rank_ref, meta_ref):
    nq = q_ref[0]
    rmn = rm_ref[0]
    mm = jnp.dot(nq.astype(jnp.bfloat16), rmn.astype(jnp.bfloat16),
                 preferred_element_type=jnp.float32)

    iota_l = lax.broadcasted_iota(jnp.int32, (L, NB), 1)
    ri = lax.broadcasted_iota(jnp.int32, (128, 128), 0)
    ci = lax.broadcasted_iota(jnp.int32, (128, 128), 1)
    tril128 = (ci <= ri).astype(jnp.float32)
    r32 = lax.broadcasted_iota(jnp.int32, (NB, NB), 0)
    c32 = lax.broadcasted_iota(jnp.int32, (NB, NB), 1)
    upper32 = (r32 < c32).astype(jnp.float32)
    cols = []
    for s in range(R):
        ms = mm[:, s * 16:(s + 1) * 16]
        sc = jnp.concatenate([ms, -ms], axis=1)
        mx = jnp.max(sc, axis=1, keepdims=True)
        h = jnp.min(jnp.where(sc == mx, iota_l, NB), axis=1, keepdims=True)
        onehot = (iota_l == h).astype(jnp.float32)
        carry = jnp.zeros((1, NB), jnp.float32)
        parts = []
        for k in range(L // 128):
            blk = onehot[k * 128:(k + 1) * 128]
            cs = jnp.dot(tril128, blk, preferred_element_type=jnp.float32) + carry
            parts.append(cs)
            carry = cs[127:128, :]
        csum = jnp.concatenate(parts, axis=0)
        hi = jnp.floor(carry * (1.0 / 256.0))
        lo = carry - hi * 256.0
        start = (jnp.dot(hi, upper32, preferred_element_type=jnp.float32)
                 * 256.0
                 + jnp.dot(lo, upper32, preferred_element_type=jnp.float32))
        rank_f = (jnp.sum(csum * onehot, axis=1, keepdims=True) - 1.0
                  + jnp.sum(onehot * start, axis=1, keepdims=True))
        rank_ref[0, s] = rank_f.astype(jnp.int32)
        cols.append(h.astype(jnp.float32))
        cols.append(jnp.floor(rank_f / BL))

    cols8 = jnp.concatenate(cols, axis=1)
    eye = _eye64()
    partsT = []
    for k in range(L // 64):
        blk = cols8[k * 64:(k + 1) * 64]
        partsT.append(lax.dot_general(blk, eye, (((0,), (0,)), ((), ())),
                                      preferred_element_type=jnp.float32))
    rows8 = jnp.concatenate(partsT, axis=1)
    meta_ref[0] = rows8.astype(jnp.int32)


def _run_k1(query, rm2d):
    return pl.pallas_call(
        _k1_body,
        grid=(B,),
        in_specs=[
            pl.BlockSpec((1, L, DK), lambda b: (b, 0, 0)),
            pl.BlockSpec((1, DK, R * 16), lambda b: (b, 0, 0)),
        ],
        out_specs=[
            pl.BlockSpec((1, R, L, 1), lambda b: (b, 0, 0, 0)),
            pl.BlockSpec((1, 8, L), lambda b: (b, 0, 0)),
        ],
        out_shape=[
            jax.ShapeDtypeStruct((B, R, L, 1), jnp.int32),
            jax.ShapeDtypeStruct((B, 8, L), jnp.int32),
        ],
    )(query, rm2d)


def _make_k2_body(soff):
    def _k2_body(qv_hbm, meta_hbm, rank_hbm, qvs_hbm, ms_hbm, qc_hbm,
                 rkfull, idxg, qvbuf, min0, min1, min2, min3,
                 mo0, mo1, mo2, mo3, qcbuf, sem):
        wid = lax.axis_index("s") * 2 + lax.axis_index("c")
        mins = [min0, min1, min2, min3]
        mouts = [mo0, mo1, mo2, mo3]

        def pair_step(t, _):
            k = wid * 2 + t
            b = k // 2
            s_ = soff + k % 2
            base = k * L
            pltpu.sync_copy(rank_hbm.at[pl.ds((b * R + s_) * L, L)], rkfull)
            pltpu.sync_copy(meta_hbm.at[pl.ds((b * 8 + 2 * s_) * L, L)], mins[0])
            for c in range(1, R):
                sp = (s_ + c) % R
                pltpu.sync_copy(meta_hbm.at[pl.ds((b * 8 + 2 * sp + 1) * L, L)],
                                mins[c])

            def mstep(i, _):
                idx = rkfull[pl.ds(i * 16, 16)]
                idx4 = idx * 4
                for cc in range(4):
                    vals = mins[cc][pl.ds(i * 16, 16)]
                    plsc.store_scatter(mouts[cc], [idx], vals)
                    plsc.store_scatter(qcbuf, [idx4 + cc], vals)
                return 0
            lax.fori_loop(0, L // 16, mstep, 0)
            for cc in range(4):
                pltpu.sync_copy(mouts[cc], ms_hbm.at[pl.ds((k * 4 + cc) * L, L)])
            pltpu.sync_copy(qcbuf, qc_hbm.at[pl.ds(k * L * 4, L * 4)])

            for c4 in range(4):
                l0 = c4 * 512
                pltpu.sync_copy(qv_hbm.at[b, pl.ds(l0, 512)], qvbuf)
                for i in range(32):
                    row, off = i // 8, (i % 8) * 16
                    idxg[row, pl.ds(off, 16)] = rkfull[pl.ds(l0 + i * 16, 16)] + base
                handles = []
                for j in range(4):
                    handles.append(pltpu.async_copy(
                        qvbuf.at[pl.ds(j * 128, 128)], qvs_hbm.at[idxg.at[j]], sem))
                for hcp in handles:
                    hcp.wait()
            return 0

        lax.fori_loop(0, 2, pair_step, 0)
    return _k2_body


def _run_k2(qv, meta, rank_flat, soff):
    mesh = plsc.VectorSubcoreMesh(core_axis_name="c", subcore_axis_name="s")
    fn = pl.kernel(
        _make_k2_body(soff),
        out_type=[
            jax.ShapeDtypeStruct((B * 2 * L, 2 * DK), jnp.float32),
            jax.ShapeDtypeStruct((B * 2 * 4 * L,), jnp.int32),
            jax.ShapeDtypeStruct((B * 2 * L * 4,), jnp.int32),
        ],
        mesh=mesh,
        compiler_params=pltpu.CompilerParams(needs_layout_passes=False),
        scratch_types=(
            [pltpu.VMEM((L,), jnp.int32),
             pltpu.VMEM((4, 128), jnp.int32),
             pltpu.VMEM((512, 2 * DK), jnp.float32)]
            + [pltpu.VMEM((L,), jnp.int32) for _ in range(8)]
            + [pltpu.VMEM((L * 4,), jnp.int32)]
            + [pltpu.SemaphoreType.DMA]
        ),
    )
    return fn(qv, meta, rank_flat)


def _k3_body(qvs_ref, ms_ref, qc_ref, att_ref):
    qvs = qvs_ref[0, 0]
    qs = qvs[:, 0:DK]
    vs = qvs[:, DK:2 * DK]
    mrows = ms_ref[0, 0].astype(jnp.float32)

    kn = qs / jnp.maximum(jnp.sqrt(jnp.sum(qs * qs, axis=1, keepdims=True)), 1e-12)
    kall = jnp.concatenate([kn[L - BL:], kn], axis=0)
    vall = jnp.concatenate([vs[L - BL:], vs], axis=0)
    mcat = jnp.concatenate([mrows[:, L - BL:], mrows], axis=1)
    qcolf = qc_ref[0, 0].astype(jnp.float32)

    inv_sqrt = 1.0 / math.sqrt(DK)
    CH = 256
    ri = lax.broadcasted_iota(jnp.int32, (CH, 2 * BL), 0)
    ci = lax.broadcasted_iota(jnp.int32, (CH, 2 * BL), 1)
    i64 = ri % BL
    later = jnp.logical_and(ci >= BL, ci - BL > i64)
    selfm = (ci - BL == i64)
    wrap0 = jnp.logical_or(later, jnp.logical_and(ri < BL, ci < BL))

    for u in range(L // CH):
        r0 = u * CH
        qk_parts = []
        for m in range(4):
            n = u * 4 + m
            qk_parts.append(lax.dot_general(
                qs[n * BL:(n + 1) * BL], kall[n * BL:n * BL + 2 * BL],
                (((1,), (1,)), ((), ())),
                preferred_element_type=jnp.float32))
        qk = jnp.concatenate(qk_parts, axis=0) * inv_sqrt
        kband = []
        for c in range(4):
            kband.append(jnp.concatenate(
                [jnp.broadcast_to(mcat[c:c + 1, m * BL:m * BL + 2 * BL], (BL, 2 * BL))
                 for m in range(u * 4, u * 4 + 4)], axis=0))
        qcol = qcolf[r0:r0 + CH]
        qk = jnp.where(qcol[:, 0:1] != kband[0], NEG_BIG, qk)
        qk = jnp.where(wrap0 if u == 0 else later, NEG_BIG, qk)
        qk = jnp.where(selfm, NEG_SELF, qk)
        ck = jnp.ones((CH, 2 * BL), jnp.float32)
        for c in range(1, 4):
            bqc = qcol[:, c:c + 1]
            prv = jnp.where(bqc == 0.0, float(NB - 1), bqc - 1.0)
            ck = (ck + (kband[c] == bqc).astype(jnp.float32)
                  + (kband[c] == prv).astype(jnp.float32))
        m_ = jnp.max(qk, axis=1, keepdims=True)
        p = jnp.exp(qk - m_)
        ssum = jnp.sum(p, axis=1, keepdims=True)
        sm = p / (ssum * ck)
        att_ref[0, 0, r0:r0 + CH, DK:DK + 1] = m_ + jnp.log(ssum)
        att_ref[0, 0, r0:r0 + CH, DK + 1:2 * DK] = jnp.zeros(
            (CH, DK - 1), jnp.float32)
        for m in range(4):
            n = u * 4 + m
            att_ref[0, 0, n * BL:(n + 1) * BL, 0:DK] = jnp.dot(
                sm[m * BL:(m + 1) * BL], vall[n * BL:n * BL + 2 * BL],
                preferred_element_type=jnp.float32)


def _run_k3(qvs, ms, qc):
    return pl.pallas_call(
        _k3_body,
        grid=(B, 2),
        in_specs=[
            pl.BlockSpec((1, 1, L, 2 * DK), lambda b, s: (b, s, 0, 0)),
            pl.BlockSpec((1, 1, 4, L), lambda b, s: (b, s, 0, 0)),
            pl.BlockSpec((1, 1, L, 4), lambda b, s: (b, s, 0, 0)),
        ],
        out_specs=pl.BlockSpec((1, 1, L, 2 * DK), lambda b, s: (b, s, 0, 0)),
        out_shape=jax.ShapeDtypeStruct((B, 2, L, 2 * DK), jnp.float32),
    )(qvs, ms, qc)


def _make_k4_body(soff):
    def _k4_body(att_hbm, rank_hbm, atto_hbm, rkfull, idxg, abuf, sem):
        wid = lax.axis_index("s") * 2 + lax.axis_index("c")

        def step2(it, _):
            t = it // 4
            c4 = it % 4
            k = wid * 2 + t
            b = k // 2
            s_ = soff + k % 2
            l0 = c4 * 512
            base = k * L
            pltpu.sync_copy(rank_hbm.at[pl.ds((b * R + s_) * L + l0, 512)], rkfull)
            for i in range(32):
                row, off = i // 8, (i % 8) * 16
                idxg[row, pl.ds(off, 16)] = rkfull[pl.ds(i * 16, 16)] + base
            handles = []
            for j in range(4):
                handles.append(pltpu.async_copy(
                    att_hbm.at[idxg.at[j]], abuf.at[pl.ds(j * 128, 128)], sem))
            for hcp in handles:
                hcp.wait()
            pltpu.sync_copy(abuf, atto_hbm.at[b, k % 2, pl.ds(l0, 512)])
            return 0

        lax.fori_loop(0, 8, step2, 0)
    return _k4_body


def _run_k4(att_flat, rank_flat, soff):
    mesh = plsc.VectorSubcoreMesh(core_axis_name="c", subcore_axis_name="s")
    fn = pl.kernel(
        _make_k4_body(soff),
        out_type=jax.ShapeDtypeStruct((B, 2, L, 2 * DK), jnp.float32),
        mesh=mesh,
        compiler_params=pltpu.CompilerParams(needs_layout_passes=False),
        scratch_types=[
            pltpu.VMEM((512,), jnp.int32),
            pltpu.VMEM((4, 128), jnp.int32),
            pltpu.VMEM((512, 2 * DK), jnp.float32),
            pltpu.SemaphoreType.DMA,
        ],
    )
    return fn(att_flat, rank_flat)


def _k5_body(atta_ref, attb_ref, out_ref):
    acc = jnp.zeros((L, DK), jnp.float32)
    for ref in (atta_ref, attb_ref):
        for s in range(2):
            col = ref[0, s, :, DK:DK + 1]
            m = jnp.max(col, axis=0, keepdims=True)
            e = jnp.exp(col - m)
            w = e / jnp.sum(e, axis=0, keepdims=True)
            acc = acc + w * ref[0, s, :, 0:DK]
    out_ref[0] = acc


def _run_k5(att_a, att_b):
    return pl.pallas_call(
        _k5_body,
        grid=(B,),
        in_specs=[pl.BlockSpec((1, 2, L, 2 * DK), lambda b: (b, 0, 0, 0)),
                  pl.BlockSpec((1, 2, L, 2 * DK), lambda b: (b, 0, 0, 0))],
        out_specs=pl.BlockSpec((1, L, DK), lambda b: (b, 0, 0)),
        out_shape=jax.ShapeDtypeStruct((B, L, DK), jnp.float32),
    )(att_a, att_b)


def kernel(query, value, seed, rand_matrix):
    qv = jnp.concatenate([query, value], axis=-1)
    nq = query / jnp.maximum(
        jnp.sqrt(jnp.sum(query * query, axis=-1, keepdims=True)), 1e-12)
    rmn = rand_matrix / jnp.sqrt(
        jnp.sum(rand_matrix * rand_matrix, axis=1, keepdims=True))
    rank, meta = _run_k1(nq, rmn.reshape(B, DK, R * 16))
    rank_flat = rank.reshape(B * R * L)
    meta_flat = meta.reshape(B * 8 * L)
    halves = []
    for soff in (0, 2):
        qvs, ms, qc = _run_k2(qv, meta_flat, rank_flat, soff)
        att = _run_k3(qvs.reshape(B, 2, L, 2 * DK), ms.reshape(B, 2, 4, L),
                      qc.reshape(B, 2, L, 4))
        halves.append(_run_k4(att.reshape(B * 2 * L, 2 * DK), rank_flat, soff))
    return _run_k5(halves[0], halves[1])
